# baseline (speedup 1.0000x reference)
import functools

import jax
import jax.numpy as jnp
from jax.experimental import pallas as pl
from jax.experimental.pallas import tpu as pltpu


def _proj_body(s1_ref, s2_ref, w_ref, hp_ref, *, nh):
    w = w_ref[...]
    hp_ref[:, :nh] = jnp.dot(s1_ref[...], w, preferred_element_type=jnp.float32)
    hp_ref[:, nh:] = jnp.dot(s2_ref[...], w, preferred_element_type=jnp.float32)


def _sweep_body(adja_ref, adjb_ref, hp_ref, b_ref, a_ref, wd_ref, bd_ref,
                sc_ref, h_s, csum_s, *, n_i, bm, nh, n, half):
    i = pl.program_id(0)

    @pl.when(i == 0)
    def _():
        csum_s[...] = jnp.zeros_like(csum_s)

    hp = hp_ref[...]
    b = b_ref[...]
    a = a_ref[...]
    pa = jnp.dot(adja_ref[...], hp, preferred_element_type=jnp.float32)
    ga = pa + b
    ha = jnp.where(ga > 0, ga, a * ga)
    h_s[pl.ds(i * bm, bm), :] = ha
    pb = jnp.dot(adjb_ref[...], hp, preferred_element_type=jnp.float32)
    gb = pb + b
    hb = jnp.where(gb > 0, gb, a * gb)
    h_s[pl.ds(half + i * bm, bm), :] = hb
    csum_s[...] += (jnp.sum(ha[:, :nh], axis=0, keepdims=True)
                    + jnp.sum(hb[:, :nh], axis=0, keepdims=True))

    @pl.when(i == n_i - 1)
    def _():
        c = jax.nn.sigmoid(csum_s[...] * (1.0 / n))
        wd = wd_ref[...]
        t1 = jnp.dot(h_s[:, :nh], wd, preferred_element_type=jnp.float32)
        t2 = jnp.dot(h_s[:, nh:], wd, preferred_element_type=jnp.float32)
        sc_ref[:, 0:1] = jnp.sum(t1 * c, axis=-1, keepdims=True) + bd_ref[...]
        sc_ref[:, 1:2] = jnp.sum(t2 * c, axis=-1, keepdims=True) + bd_ref[...]


def kernel(seq1, seq2, adj, sparse, W_fc, b_gcn, a_prelu, W_disc, b_disc):
    n = seq1.shape[1]
    nin = W_fc.shape[0]
    nh = W_fc.shape[1]
    s1 = seq1.reshape(n, nin)
    s2 = seq2.reshape(n, nin)
    a2 = adj.reshape(n, n)
    b2 = jnp.concatenate([b_gcn, b_gcn]).reshape(1, 2 * nh)
    a_p = jnp.asarray(a_prelu, jnp.float32).reshape(1, 1)
    bd = jnp.asarray(b_disc, jnp.float32).reshape(1, 1)

    bp = 2000
    bm = 200
    half = n // 2
    n_i = half // bm
    hblk = half // bm

    hp = pl.pallas_call(
        functools.partial(_proj_body, nh=nh),
        grid=(n // bp,),
        in_specs=[
            pl.BlockSpec((bp, nin), lambda p: (p, 0)),
            pl.BlockSpec((bp, nin), lambda p: (p, 0)),
            pl.BlockSpec((nin, nh), lambda p: (0, 0)),
        ],
        out_specs=pl.BlockSpec((bp, 2 * nh), lambda p: (p, 0)),
        out_shape=jax.ShapeDtypeStruct((n, 2 * nh), jnp.float32),
        compiler_params=pltpu.CompilerParams(
            dimension_semantics=("arbitrary",),
        ),
    )(s1, s2, W_fc)

    sc = pl.pallas_call(
        functools.partial(_sweep_body, n_i=n_i, bm=bm, nh=nh, n=float(n), half=half),
        grid=(n_i,),
        in_specs=[
            pl.BlockSpec((bm, n), lambda i: (i, 0)),
            pl.BlockSpec((bm, n), lambda i: (i + hblk, 0)),
            pl.BlockSpec((n, 2 * nh), lambda i: (0, 0)),
            pl.BlockSpec((1, 2 * nh), lambda i: (0, 0)),
            pl.BlockSpec((1, 1), lambda i: (0, 0)),
            pl.BlockSpec((nh, nh), lambda i: (0, 0)),
            pl.BlockSpec((1, 1), lambda i: (0, 0)),
        ],
        out_specs=pl.BlockSpec((n, 2), lambda i: (0, 0)),
        out_shape=jax.ShapeDtypeStruct((n, 2), jnp.float32),
        scratch_shapes=[
            pltpu.VMEM((n, 2 * nh), jnp.float32),
            pltpu.VMEM((1, nh), jnp.float32),
        ],
        compiler_params=pltpu.CompilerParams(
            dimension_semantics=("arbitrary",),
            vmem_limit_bytes=64 * 1024 * 1024,
        ),
    )(a2, a2, hp, b2, a_p, W_disc, bd)

    return jnp.concatenate([sc[:, 0].reshape(1, n), sc[:, 1].reshape(1, n)], axis=1)
